# baseline (speedup 1.0000x reference)
import functools

import jax
import jax.numpy as jnp
from jax import lax
from jax.experimental import pallas as pl
from jax.experimental.pallas import tpu as pltpu
from jax.experimental.pallas import tpu_sc as plsc

N = 10000
NPAD = 10240
K = 16
DT = 128
NEG = 0.01
BA = 512
BB = 256
BD = 320
NW = 32
RPW = NPAD // NW
GCH = 128

_PCALL = functools.partial(pl.pallas_call)


def _bdot(a, b):
    return lax.dot_general(a.astype(jnp.bfloat16), b.astype(jnp.bfloat16),
                           (((1,), (0,)), ((), ())),
                           preferred_element_type=jnp.float32)


def _leaky(x):
    return jnp.where(x >= 0, x, NEG * x)


def _a_body(x_ref, w1, b1, w2, b2, w3, b3, w4, b4, ws, bs, wh, bh, wo1,
            s_ref, tab_ref, p1_ref):
    h = x_ref[...]
    for wt, bt in ((w1, b1), (w2, b2), (w3, b3), (w4, b4)):
        h = _leaky(_bdot(h, wt[...]) + bt[...])
    s = _bdot(h, ws[...]) + bs[...]
    hf = _bdot(h, wh[...]) + bh[...]
    s_ref[...] = s
    p1_ref[...] = _bdot(h, wo1[...])
    tab_ref[...] = jnp.concatenate(
        [hf, s, jnp.zeros((h.shape[0], DT - 34), jnp.float32)], axis=1)


def _run_a(xp, wts):
    full = [pl.BlockSpec(w.shape, lambda i: (0,) * w.ndim) for w in wts]
    return _PCALL(
        _a_body,
        grid=(NPAD // BA,),
        in_specs=[pl.BlockSpec((BA, 128), lambda i: (i, 0))] + full,
        out_specs=[pl.BlockSpec((BA, 2), lambda i: (i, 0)),
                   pl.BlockSpec((BA, DT), lambda i: (i, 0)),
                   pl.BlockSpec((BA, 32), lambda i: (i, 0))],
        out_shape=[jax.ShapeDtypeStruct((NPAD, 2), jnp.float32),
                   jax.ShapeDtypeStruct((NPAD, DT), jnp.float32),
                   jax.ShapeDtypeStruct((NPAD, 32), jnp.float32)],
    )(xp, *wts)


def _b_body(s_ref, sct_ref, nbr_ref):
    r0 = s_ref[:, 0:1]
    r1 = s_ref[:, 1:2]
    c0 = sct_ref[0:1, :]
    c1 = sct_ref[1:2, :]
    sq_r = r0 * r0 + r1 * r1
    sq_c = c0 * c0 + c1 * c1
    rb0 = r0.astype(jnp.bfloat16).astype(jnp.float32)
    rb1 = r1.astype(jnp.bfloat16).astype(jnp.float32)
    cb0 = c0.astype(jnp.bfloat16).astype(jnp.float32)
    cb1 = c1.astype(jnp.bfloat16).astype(jnp.float32)
    dot = rb0 * cb0 + rb1 * cb1
    d2 = (sq_r + sq_c) - 2.0 * dot
    iota_c = lax.broadcasted_iota(jnp.int32, (1, NPAD), 1)
    d2 = jnp.where(iota_c >= N, jnp.inf, d2)
    for j in range(K):
        m = jnp.min(d2, axis=1, keepdims=True)
        idx = jnp.min(jnp.where(d2 <= m, iota_c, NPAD), axis=1, keepdims=True)
        nbr_ref[:, j:j + 1] = idx
        d2 = jnp.where((d2 <= m) & (iota_c == idx), jnp.inf, d2)


def _run_b(s, sct):
    return _PCALL(
        _b_body,
        grid=(NPAD // BB,),
        in_specs=[pl.BlockSpec((BB, 2), lambda i: (i, 0)),
                  pl.BlockSpec((2, NPAD), lambda i: (0, 0))],
        out_specs=pl.BlockSpec((BB, K), lambda i: (i, 0)),
        out_shape=jax.ShapeDtypeStruct((NPAD, K), jnp.int32),
    )(s, sct)


EPW = K * RPW
NCH = EPW // GCH
NBUF = 4


@functools.cache
def _sc_gather_fn():
    @functools.partial(
        pl.kernel,
        mesh=plsc.VectorSubcoreMesh(core_axis_name="c", subcore_axis_name="s"),
        out_type=jax.ShapeDtypeStruct((NW, EPW, DT), jnp.float32),
        scratch_types=[pltpu.VMEM((EPW,), jnp.int32),
                       pltpu.VMEM((NBUF, GCH, DT), jnp.float32),
                       pltpu.SemaphoreType.DMA,
                       pltpu.SemaphoreType.DMA],
    )
    def _sc_gather(tab_hbm, nbrw_hbm, out_hbm, idx_v, bufs_v, gsem, wsem):
        wid = lax.axis_index("s") * 2 + lax.axis_index("c")
        pltpu.sync_copy(nbrw_hbm.at[pl.ds(wid * EPW, EPW)], idx_v)
        for grp in range(NCH // NBUF):
            gds = []
            for u in range(NBUF):
                t = grp * NBUF + u
                gds.append(pltpu.async_copy(
                    tab_hbm.at[idx_v.at[pl.ds(t * GCH, GCH)]],
                    bufs_v.at[u], gsem))
            wds = []
            for u in range(NBUF):
                t = grp * NBUF + u
                gds[u].wait()
                wds.append(pltpu.async_copy(
                    bufs_v.at[u], out_hbm.at[wid, pl.ds(t * GCH, GCH)], wsem))
            for u in range(NBUF):
                wds[u].wait()

    return _sc_gather


def _d_body(gath_ref, s_ref, p1_ref, x47_ref,
            wo2, bo2, n2w1, n2b1, n2w2, n2b2, n2w3, n2b3, n2w4, n2b4,
            n3w1, n3b1, n3w2, n3b2, n3w3, n3b3, n3w4, n3b4,
            ids_ref, p4_ref):
    s = s_ref[...]
    s0 = s[:, 0:1]
    s1 = s[:, 1:2]
    acc = jnp.zeros((BD, 32), jnp.float32)
    mx = jnp.full((BD, 32), -jnp.inf, jnp.float32)
    for j in range(K):
        g = gath_ref[0, j * RPW:(j + 1) * RPW, :]
        hj = g[:, 0:32]
        e0 = (s0 - g[:, 32:33]) ** 2
        e1 = (s1 - g[:, 33:34]) ** 2
        w = jnp.exp(-10.0 * (e0 + e1))
        msg = hj * w
        acc = acc + msg
        mx = jnp.maximum(mx, msg)
    agg = jnp.concatenate([acc * (1.0 / K), mx], axis=1)
    h2 = _leaky((p1_ref[...] + _bdot(agg, wo2[...])) + bo2[...])
    t = h2
    for wt, bt in ((n2w1, n2b1), (n2w2, n2b2), (n2w3, n2b3)):
        t = _leaky(_bdot(t, wt[...]) + bt[...])
    ids = _bdot(t, n2w4[...]) + n2b4[...]
    u = jnp.concatenate([h2, ids], axis=1)
    for wt, bt in ((n3w1, n3b1), (n3w2, n3b2), (n3w3, n3b3)):
        u = _leaky(_bdot(u, wt[...]) + bt[...])
    p4 = x47_ref[...] + (_bdot(u, n3w4[...]) + n3b4[...])
    ids_ref[...] = ids
    p4_ref[...] = p4


def _run_d(gath, s, p1, x47, wts):
    full = [pl.BlockSpec(w.shape, lambda i: (0,) * w.ndim) for w in wts]
    return _PCALL(
        _d_body,
        grid=(NPAD // BD,),
        in_specs=[pl.BlockSpec((1, EPW, DT), lambda i: (i, 0, 0)),
                  pl.BlockSpec((BD, 2), lambda i: (i, 0)),
                  pl.BlockSpec((BD, 32), lambda i: (i, 0)),
                  pl.BlockSpec((BD, 4), lambda i: (i, 0))] + full,
        out_specs=[pl.BlockSpec((BD, 6), lambda i: (i, 0)),
                   pl.BlockSpec((BD, 4), lambda i: (i, 0))],
        out_shape=[jax.ShapeDtypeStruct((NPAD, 6), jnp.float32),
                   jax.ShapeDtypeStruct((NPAD, 4), jnp.float32)],
    )(gath, s, p1, x47, *wts)


def _wt(W):
    return jnp.transpose(W)


def _bt(b):
    return jnp.reshape(b, (1, -1))


def kernel(x, nn1, conv, nn2, nn3):
    Ws, bs, Wh, bh, Wo1, Wo2, bo2 = conv
    xp = jnp.pad(x, ((0, NPAD - N), (0, 0)))

    a_wts = []
    for (W, b) in nn1:
        a_wts += [_wt(W), _bt(b)]
    a_wts += [_wt(Ws), _bt(bs), _wt(Wh), _bt(bh), _wt(Wo1)]
    s, tab, p1 = _run_a(xp, a_wts)

    nbr = _run_b(s, jnp.transpose(s))
    nbrw = jnp.reshape(
        jnp.transpose(jnp.reshape(jnp.transpose(nbr), (K, NW, RPW)), (1, 0, 2)),
        (-1,))
    gath = _sc_gather_fn()(tab, nbrw)

    d_wts = [_wt(Wo2), _bt(bo2)]
    for (W, b) in nn2:
        d_wts += [_wt(W), _bt(b)]
    for (W, b) in nn3:
        d_wts += [_wt(W), _bt(b)]
    x47 = xp[:, 3:7]
    ids, p4 = _run_d(gath, s, p1, x47, d_wts)
    return (ids[:N], p4[:N])


# speedup vs baseline: 1.1492x; 1.1492x over previous
import functools

import jax
import jax.numpy as jnp
from jax import lax
from jax.experimental import pallas as pl
from jax.experimental.pallas import tpu as pltpu
from jax.experimental.pallas import tpu_sc as plsc

N = 10000
NPAD = 10240
K = 16
DT = 128
NEG = 0.01
BA = 512
BB = 512
BD = 320
NW = 32
RPW = NPAD // NW
GCH = 128

_PCALL = functools.partial(pl.pallas_call)


def _bdot(a, b):
    return lax.dot_general(a.astype(jnp.bfloat16), b.astype(jnp.bfloat16),
                           (((1,), (0,)), ((), ())),
                           preferred_element_type=jnp.float32)


def _leaky(x):
    return jnp.where(x >= 0, x, NEG * x)


def _a_body(x_ref, w1, b1, w2, b2, w3, b3, w4, b4, ws, bs, wh, bh, wo1,
            s_ref, tab_ref, p1_ref):
    h = x_ref[...]
    for wt, bt in ((w1, b1), (w2, b2), (w3, b3), (w4, b4)):
        h = _leaky(_bdot(h, wt[...]) + bt[...])
    s = _bdot(h, ws[...]) + bs[...]
    hf = _bdot(h, wh[...]) + bh[...]
    s_ref[...] = s
    p1_ref[...] = _bdot(h, wo1[...])
    tab_ref[...] = jnp.concatenate(
        [hf, s, jnp.zeros((h.shape[0], DT - 34), jnp.float32)], axis=1)


def _run_a(xp, wts):
    full = [pl.BlockSpec(w.shape, lambda i: (0,) * w.ndim) for w in wts]
    return _PCALL(
        _a_body,
        grid=(NPAD // BA,),
        in_specs=[pl.BlockSpec((BA, 128), lambda i: (i, 0))] + full,
        out_specs=[pl.BlockSpec((BA, 2), lambda i: (i, 0)),
                   pl.BlockSpec((BA, DT), lambda i: (i, 0)),
                   pl.BlockSpec((BA, 32), lambda i: (i, 0))],
        out_shape=[jax.ShapeDtypeStruct((NPAD, 2), jnp.float32),
                   jax.ShapeDtypeStruct((NPAD, DT), jnp.float32),
                   jax.ShapeDtypeStruct((NPAD, 32), jnp.float32)],
    )(xp, *wts)


def _b_body(s_ref, sct_ref, nbr_ref):
    r0 = s_ref[:, 0:1]
    r1 = s_ref[:, 1:2]
    c0 = sct_ref[0:1, :]
    c1 = sct_ref[1:2, :]
    sq_r = r0 * r0 + r1 * r1
    sq_c = c0 * c0 + c1 * c1
    rb0 = r0.astype(jnp.bfloat16).astype(jnp.float32)
    rb1 = r1.astype(jnp.bfloat16).astype(jnp.float32)
    cb0 = c0.astype(jnp.bfloat16).astype(jnp.float32)
    cb1 = c1.astype(jnp.bfloat16).astype(jnp.float32)
    dot = rb0 * cb0 + rb1 * cb1
    d2 = (sq_r + sq_c) - 2.0 * dot
    iota_c = lax.broadcasted_iota(jnp.int32, (1, NPAD), 1)
    d2 = jnp.where(iota_c >= N, jnp.inf, d2)
    for j in range(K):
        m = jnp.min(d2, axis=1, keepdims=True)
        cand = jnp.where(d2 <= m, iota_c, NPAD)
        idx = jnp.min(cand, axis=1, keepdims=True)
        nbr_ref[:, j:j + 1] = idx
        d2 = jnp.where(cand == idx, jnp.inf, d2)


def _run_b(s, sct):
    return _PCALL(
        _b_body,
        grid=(NPAD // BB,),
        in_specs=[pl.BlockSpec((BB, 2), lambda i: (i, 0)),
                  pl.BlockSpec((2, NPAD), lambda i: (0, 0))],
        out_specs=pl.BlockSpec((BB, K), lambda i: (i, 0)),
        out_shape=jax.ShapeDtypeStruct((NPAD, K), jnp.int32),
    )(s, sct)


EPW = K * RPW
NCH = EPW // GCH
NBUF = 4


@functools.cache
def _sc_gather_fn():
    @functools.partial(
        pl.kernel,
        mesh=plsc.VectorSubcoreMesh(core_axis_name="c", subcore_axis_name="s"),
        out_type=jax.ShapeDtypeStruct((NW, EPW, DT), jnp.float32),
        scratch_types=[pltpu.VMEM((EPW,), jnp.int32),
                       pltpu.VMEM((NBUF, GCH, DT), jnp.float32),
                       pltpu.SemaphoreType.DMA,
                       pltpu.SemaphoreType.DMA],
    )
    def _sc_gather(tab_hbm, nbrw_hbm, out_hbm, idx_v, bufs_v, gsem, wsem):
        wid = lax.axis_index("s") * 2 + lax.axis_index("c")
        pltpu.sync_copy(nbrw_hbm.at[pl.ds(wid * EPW, EPW)], idx_v)
        for grp in range(NCH // NBUF):
            gds = []
            for u in range(NBUF):
                t = grp * NBUF + u
                gds.append(pltpu.async_copy(
                    tab_hbm.at[idx_v.at[pl.ds(t * GCH, GCH)]],
                    bufs_v.at[u], gsem))
            wds = []
            for u in range(NBUF):
                t = grp * NBUF + u
                gds[u].wait()
                wds.append(pltpu.async_copy(
                    bufs_v.at[u], out_hbm.at[wid, pl.ds(t * GCH, GCH)], wsem))
            for u in range(NBUF):
                wds[u].wait()

    return _sc_gather


def _d_body(gath_ref, s_ref, p1_ref, x47_ref,
            wo2, bo2, n2w1, n2b1, n2w2, n2b2, n2w3, n2b3, n2w4, n2b4,
            n3w1, n3b1, n3w2, n3b2, n3w3, n3b3, n3w4, n3b4,
            ids_ref, p4_ref):
    s = s_ref[...]
    s0 = s[:, 0:1]
    s1 = s[:, 1:2]
    acc = jnp.zeros((BD, 32), jnp.float32)
    mx = jnp.full((BD, 32), -jnp.inf, jnp.float32)
    for j in range(K):
        g = gath_ref[0, j * RPW:(j + 1) * RPW, :]
        hj = g[:, 0:32]
        e0 = (s0 - g[:, 32:33]) ** 2
        e1 = (s1 - g[:, 33:34]) ** 2
        w = jnp.exp(-10.0 * (e0 + e1))
        msg = hj * w
        acc = acc + msg
        mx = jnp.maximum(mx, msg)
    agg = jnp.concatenate([acc * (1.0 / K), mx], axis=1)
    h2 = _leaky((p1_ref[...] + _bdot(agg, wo2[...])) + bo2[...])
    t = h2
    for wt, bt in ((n2w1, n2b1), (n2w2, n2b2), (n2w3, n2b3)):
        t = _leaky(_bdot(t, wt[...]) + bt[...])
    ids = _bdot(t, n2w4[...]) + n2b4[...]
    u = jnp.concatenate([h2, ids], axis=1)
    for wt, bt in ((n3w1, n3b1), (n3w2, n3b2), (n3w3, n3b3)):
        u = _leaky(_bdot(u, wt[...]) + bt[...])
    p4 = x47_ref[...] + (_bdot(u, n3w4[...]) + n3b4[...])
    ids_ref[...] = ids
    p4_ref[...] = p4


def _run_d(gath, s, p1, x47, wts):
    full = [pl.BlockSpec(w.shape, lambda i: (0,) * w.ndim) for w in wts]
    return _PCALL(
        _d_body,
        grid=(NPAD // BD,),
        in_specs=[pl.BlockSpec((1, EPW, DT), lambda i: (i, 0, 0)),
                  pl.BlockSpec((BD, 2), lambda i: (i, 0)),
                  pl.BlockSpec((BD, 32), lambda i: (i, 0)),
                  pl.BlockSpec((BD, 4), lambda i: (i, 0))] + full,
        out_specs=[pl.BlockSpec((BD, 6), lambda i: (i, 0)),
                   pl.BlockSpec((BD, 4), lambda i: (i, 0))],
        out_shape=[jax.ShapeDtypeStruct((NPAD, 6), jnp.float32),
                   jax.ShapeDtypeStruct((NPAD, 4), jnp.float32)],
    )(gath, s, p1, x47, *wts)


def _wt(W):
    return jnp.transpose(W)


def _bt(b):
    return jnp.reshape(b, (1, -1))


def kernel(x, nn1, conv, nn2, nn3):
    Ws, bs, Wh, bh, Wo1, Wo2, bo2 = conv
    xp = jnp.pad(x, ((0, NPAD - N), (0, 0)))

    a_wts = []
    for (W, b) in nn1:
        a_wts += [_wt(W), _bt(b)]
    a_wts += [_wt(Ws), _bt(bs), _wt(Wh), _bt(bh), _wt(Wo1)]
    s, tab, p1 = _run_a(xp, a_wts)

    nbr = _run_b(s, jnp.transpose(s))
    nbrw = jnp.reshape(
        jnp.transpose(jnp.reshape(jnp.transpose(nbr), (K, NW, RPW)), (1, 0, 2)),
        (-1,))
    gath = _sc_gather_fn()(tab, nbrw)

    d_wts = [_wt(Wo2), _bt(bo2)]
    for (W, b) in nn2:
        d_wts += [_wt(W), _bt(b)]
    for (W, b) in nn3:
        d_wts += [_wt(W), _bt(b)]
    x47 = xp[:, 3:7]
    ids, p4 = _run_d(gath, s, p1, x47, d_wts)
    return (ids[:N], p4[:N])


# speedup vs baseline: 1.1783x; 1.0253x over previous
import functools

import jax
import jax.numpy as jnp
from jax import lax
from jax.experimental import pallas as pl
from jax.experimental.pallas import tpu as pltpu
from jax.experimental.pallas import tpu_sc as plsc

N = 10000
NPAD = 10240
K = 16
DT = 128
NEG = 0.01
BA = 512
BB = 512
BD = 320
NW = 32
RPW = NPAD // NW
GCH = 128

_PCALL = functools.partial(pl.pallas_call)


def _bdot(a, b):
    return lax.dot_general(a.astype(jnp.bfloat16), b.astype(jnp.bfloat16),
                           (((1,), (0,)), ((), ())),
                           preferred_element_type=jnp.float32)


def _leaky(x):
    return jnp.where(x >= 0, x, NEG * x)


def _a_body(x_ref, w1, b1, w2, b2, w3, b3, w4, b4, ws, bs, wh, bh, wo1,
            s_ref, tab_ref, p1_ref):
    h = x_ref[...]
    for wt, bt in ((w1, b1), (w2, b2), (w3, b3), (w4, b4)):
        h = _leaky(_bdot(h, wt[...]) + bt[...])
    s = _bdot(h, ws[...]) + bs[...]
    hf = _bdot(h, wh[...]) + bh[...]
    s_ref[...] = s
    p1_ref[...] = _bdot(h, wo1[...])
    tab_ref[...] = jnp.concatenate(
        [hf, s, jnp.zeros((h.shape[0], DT - 34), jnp.float32)], axis=1)


def _run_a(xp, wts):
    full = [pl.BlockSpec(w.shape, lambda i: (0,) * w.ndim) for w in wts]
    return _PCALL(
        _a_body,
        grid=(NPAD // BA,),
        in_specs=[pl.BlockSpec((BA, 128), lambda i: (i, 0))] + full,
        out_specs=[pl.BlockSpec((BA, 2), lambda i: (i, 0)),
                   pl.BlockSpec((BA, DT), lambda i: (i, 0)),
                   pl.BlockSpec((BA, 32), lambda i: (i, 0))],
        out_shape=[jax.ShapeDtypeStruct((NPAD, 2), jnp.float32),
                   jax.ShapeDtypeStruct((NPAD, DT), jnp.float32),
                   jax.ShapeDtypeStruct((NPAD, 32), jnp.float32)],
    )(xp, *wts)


def _b_body(s_ref, sct_ref, nbr_ref):
    r0 = s_ref[:, 0:1]
    r1 = s_ref[:, 1:2]
    c0 = sct_ref[0:1, :]
    c1 = sct_ref[1:2, :]
    sq_r = r0 * r0 + r1 * r1
    sq_c = c0 * c0 + c1 * c1
    rb0 = r0.astype(jnp.bfloat16).astype(jnp.float32)
    rb1 = r1.astype(jnp.bfloat16).astype(jnp.float32)
    cb0 = c0.astype(jnp.bfloat16).astype(jnp.float32)
    cb1 = c1.astype(jnp.bfloat16).astype(jnp.float32)
    dot = rb0 * cb0 + rb1 * cb1
    d2 = (sq_r + sq_c) - 2.0 * dot
    iota_c = lax.broadcasted_iota(jnp.int32, (1, NPAD), 1)
    d2 = jnp.where(iota_c >= N, jnp.inf, d2)
    for j in range(K):
        idx = jnp.argmin(d2, axis=1).astype(jnp.int32)[:, None]
        nbr_ref[:, j:j + 1] = idx
        d2 = jnp.where(iota_c == idx, jnp.inf, d2)


def _run_b(s, sct):
    return _PCALL(
        _b_body,
        grid=(NPAD // BB,),
        in_specs=[pl.BlockSpec((BB, 2), lambda i: (i, 0)),
                  pl.BlockSpec((2, NPAD), lambda i: (0, 0))],
        out_specs=pl.BlockSpec((BB, K), lambda i: (i, 0)),
        out_shape=jax.ShapeDtypeStruct((NPAD, K), jnp.int32),
    )(s, sct)


EPW = K * RPW
NCH = EPW // GCH
NBUF = 4


@functools.cache
def _sc_gather_fn():
    @functools.partial(
        pl.kernel,
        mesh=plsc.VectorSubcoreMesh(core_axis_name="c", subcore_axis_name="s"),
        out_type=jax.ShapeDtypeStruct((NW, EPW, DT), jnp.float32),
        scratch_types=[pltpu.VMEM((EPW,), jnp.int32),
                       pltpu.VMEM((NBUF, GCH, DT), jnp.float32),
                       pltpu.SemaphoreType.DMA,
                       pltpu.SemaphoreType.DMA],
    )
    def _sc_gather(tab_hbm, nbrw_hbm, out_hbm, idx_v, bufs_v, gsem, wsem):
        wid = lax.axis_index("s") * 2 + lax.axis_index("c")
        pltpu.sync_copy(nbrw_hbm.at[pl.ds(wid * EPW, EPW)], idx_v)
        for grp in range(NCH // NBUF):
            gds = []
            for u in range(NBUF):
                t = grp * NBUF + u
                gds.append(pltpu.async_copy(
                    tab_hbm.at[idx_v.at[pl.ds(t * GCH, GCH)]],
                    bufs_v.at[u], gsem))
            wds = []
            for u in range(NBUF):
                t = grp * NBUF + u
                gds[u].wait()
                wds.append(pltpu.async_copy(
                    bufs_v.at[u], out_hbm.at[wid, pl.ds(t * GCH, GCH)], wsem))
            for u in range(NBUF):
                wds[u].wait()

    return _sc_gather


def _d_body(gath_ref, s_ref, p1_ref, x47_ref,
            wo2, bo2, n2w1, n2b1, n2w2, n2b2, n2w3, n2b3, n2w4, n2b4,
            n3w1, n3b1, n3w2, n3b2, n3w3, n3b3, n3w4, n3b4,
            ids_ref, p4_ref):
    s = s_ref[...]
    s0 = s[:, 0:1]
    s1 = s[:, 1:2]
    acc = jnp.zeros((BD, 32), jnp.float32)
    mx = jnp.full((BD, 32), -jnp.inf, jnp.float32)
    for j in range(K):
        g = gath_ref[0, j * RPW:(j + 1) * RPW, :]
        hj = g[:, 0:32]
        e0 = (s0 - g[:, 32:33]) ** 2
        e1 = (s1 - g[:, 33:34]) ** 2
        w = jnp.exp(-10.0 * (e0 + e1))
        msg = hj * w
        acc = acc + msg
        mx = jnp.maximum(mx, msg)
    agg = jnp.concatenate([acc * (1.0 / K), mx], axis=1)
    h2 = _leaky((p1_ref[...] + _bdot(agg, wo2[...])) + bo2[...])
    t = h2
    for wt, bt in ((n2w1, n2b1), (n2w2, n2b2), (n2w3, n2b3)):
        t = _leaky(_bdot(t, wt[...]) + bt[...])
    ids = _bdot(t, n2w4[...]) + n2b4[...]
    u = jnp.concatenate([h2, ids], axis=1)
    for wt, bt in ((n3w1, n3b1), (n3w2, n3b2), (n3w3, n3b3)):
        u = _leaky(_bdot(u, wt[...]) + bt[...])
    p4 = x47_ref[...] + (_bdot(u, n3w4[...]) + n3b4[...])
    ids_ref[...] = ids
    p4_ref[...] = p4


def _run_d(gath, s, p1, x47, wts):
    full = [pl.BlockSpec(w.shape, lambda i: (0,) * w.ndim) for w in wts]
    return _PCALL(
        _d_body,
        grid=(NPAD // BD,),
        in_specs=[pl.BlockSpec((1, EPW, DT), lambda i: (i, 0, 0)),
                  pl.BlockSpec((BD, 2), lambda i: (i, 0)),
                  pl.BlockSpec((BD, 32), lambda i: (i, 0)),
                  pl.BlockSpec((BD, 4), lambda i: (i, 0))] + full,
        out_specs=[pl.BlockSpec((BD, 6), lambda i: (i, 0)),
                   pl.BlockSpec((BD, 4), lambda i: (i, 0))],
        out_shape=[jax.ShapeDtypeStruct((NPAD, 6), jnp.float32),
                   jax.ShapeDtypeStruct((NPAD, 4), jnp.float32)],
    )(gath, s, p1, x47, *wts)


def _wt(W):
    return jnp.transpose(W)


def _bt(b):
    return jnp.reshape(b, (1, -1))


def kernel(x, nn1, conv, nn2, nn3):
    Ws, bs, Wh, bh, Wo1, Wo2, bo2 = conv
    xp = jnp.pad(x, ((0, NPAD - N), (0, 0)))

    a_wts = []
    for (W, b) in nn1:
        a_wts += [_wt(W), _bt(b)]
    a_wts += [_wt(Ws), _bt(bs), _wt(Wh), _bt(bh), _wt(Wo1)]
    s, tab, p1 = _run_a(xp, a_wts)

    nbr = _run_b(s, jnp.transpose(s))
    nbrw = jnp.reshape(
        jnp.transpose(jnp.reshape(jnp.transpose(nbr), (K, NW, RPW)), (1, 0, 2)),
        (-1,))
    gath = _sc_gather_fn()(tab, nbrw)

    d_wts = [_wt(Wo2), _bt(bo2)]
    for (W, b) in nn2:
        d_wts += [_wt(W), _bt(b)]
    for (W, b) in nn3:
        d_wts += [_wt(W), _bt(b)]
    x47 = xp[:, 3:7]
    ids, p4 = _run_d(gath, s, p1, x47, d_wts)
    return (ids[:N], p4[:N])


# speedup vs baseline: 1.2591x; 1.0686x over previous
import functools

import jax
import jax.numpy as jnp
from jax import lax
from jax.experimental import pallas as pl
from jax.experimental.pallas import tpu as pltpu
from jax.experimental.pallas import tpu_sc as plsc

N = 10000
NPAD = 10240
K = 16
DT = 128
NEG = 0.01
BA = 512
BB = 512
BD = 320
NW = 32
RPW = NPAD // NW
GCH = 128

_PCALL = functools.partial(pl.pallas_call)


def _bdot(a, b):
    return lax.dot_general(a.astype(jnp.bfloat16), b.astype(jnp.bfloat16),
                           (((1,), (0,)), ((), ())),
                           preferred_element_type=jnp.float32)


def _leaky(x):
    return jnp.where(x >= 0, x, NEG * x)


def _a_body(x_ref, w1, b1, w2, b2, w3, b3, w4, b4, ws, bs, wh, bh, wo1,
            s_ref, tab_ref, p1_ref):
    h = x_ref[...]
    for wt, bt in ((w1, b1), (w2, b2), (w3, b3), (w4, b4)):
        h = _leaky(_bdot(h, wt[...]) + bt[...])
    s = _bdot(h, ws[...]) + bs[...]
    hf = _bdot(h, wh[...]) + bh[...]
    s_ref[...] = s
    p1_ref[...] = _bdot(h, wo1[...])
    tab_ref[...] = jnp.concatenate(
        [hf, s, jnp.zeros((h.shape[0], DT - 34), jnp.float32)], axis=1)


def _run_a(xp, wts):
    full = [pl.BlockSpec(w.shape, lambda i: (0,) * w.ndim) for w in wts]
    return _PCALL(
        _a_body,
        grid=(NPAD // BA,),
        in_specs=[pl.BlockSpec((BA, 128), lambda i: (i, 0))] + full,
        out_specs=[pl.BlockSpec((BA, 2), lambda i: (i, 0)),
                   pl.BlockSpec((BA, DT), lambda i: (i, 0)),
                   pl.BlockSpec((BA, 32), lambda i: (i, 0))],
        out_shape=[jax.ShapeDtypeStruct((NPAD, 2), jnp.float32),
                   jax.ShapeDtypeStruct((NPAD, DT), jnp.float32),
                   jax.ShapeDtypeStruct((NPAD, 32), jnp.float32)],
    )(xp, *wts)


def _b_body(s_ref, sct_ref, nbr_ref):
    r0 = s_ref[:, 0:1]
    r1 = s_ref[:, 1:2]
    c0 = sct_ref[0:1, :]
    c1 = sct_ref[1:2, :]
    sq_r = r0 * r0 + r1 * r1
    sq_c = c0 * c0 + c1 * c1
    rb0 = r0.astype(jnp.bfloat16).astype(jnp.float32)
    rb1 = r1.astype(jnp.bfloat16).astype(jnp.float32)
    cb0 = c0.astype(jnp.bfloat16).astype(jnp.float32)
    cb1 = c1.astype(jnp.bfloat16).astype(jnp.float32)
    dot = rb0 * cb0 + rb1 * cb1
    d2 = (sq_r + sq_c) - 2.0 * dot
    iota_c = lax.broadcasted_iota(jnp.int32, (1, NPAD), 1)
    d2 = jnp.where(iota_c >= N, jnp.inf, d2)
    for j in range(K):
        idx = jnp.argmin(d2, axis=1).astype(jnp.int32)[:, None]
        nbr_ref[:, j:j + 1] = idx
        d2 = jnp.where(iota_c == idx, jnp.inf, d2)


def _run_b(s_half, sct, nh):
    return _PCALL(
        _b_body,
        grid=(nh // BB,),
        in_specs=[pl.BlockSpec((BB, 2), lambda i: (i, 0)),
                  pl.BlockSpec((2, NPAD), lambda i: (0, 0))],
        out_specs=pl.BlockSpec((BB, K), lambda i: (i, 0)),
        out_shape=jax.ShapeDtypeStruct((nh, K), jnp.int32),
    )(s_half, sct)


NBUF = 5


@functools.cache
def _sc_gather_fn(epw):
    nch = epw // GCH

    @functools.partial(
        pl.kernel,
        mesh=plsc.VectorSubcoreMesh(core_axis_name="c", subcore_axis_name="s"),
        out_type=jax.ShapeDtypeStruct((NW, epw, DT), jnp.float32),
        scratch_types=[pltpu.VMEM((epw,), jnp.int32),
                       pltpu.VMEM((NBUF, GCH, DT), jnp.float32),
                       pltpu.SemaphoreType.DMA,
                       pltpu.SemaphoreType.DMA],
    )
    def _sc_gather(tab_hbm, nbrw_hbm, out_hbm, idx_v, bufs_v, gsem, wsem):
        wid = lax.axis_index("s") * 2 + lax.axis_index("c")
        pltpu.sync_copy(nbrw_hbm.at[pl.ds(wid * epw, epw)], idx_v)
        for grp in range(-(-nch // NBUF)):
            gds = []
            for u in range(NBUF):
                t = grp * NBUF + u
                if t >= nch:
                    break
                gds.append(pltpu.async_copy(
                    tab_hbm.at[idx_v.at[pl.ds(t * GCH, GCH)]],
                    bufs_v.at[u], gsem))
            wds = []
            for u in range(len(gds)):
                t = grp * NBUF + u
                gds[u].wait()
                wds.append(pltpu.async_copy(
                    bufs_v.at[u], out_hbm.at[wid, pl.ds(t * GCH, GCH)], wsem))
            for u in range(len(wds)):
                wds[u].wait()

    return _sc_gather


def _d_body(gath_ref, s_ref, p1_ref, x47_ref,
            wo2, bo2, n2w1, n2b1, n2w2, n2b2, n2w3, n2b3, n2w4, n2b4,
            n3w1, n3b1, n3w2, n3b2, n3w3, n3b3, n3w4, n3b4,
            ids_ref, p4_ref):
    rpw = s_ref.shape[0]
    s = s_ref[...]
    s0 = s[:, 0:1]
    s1 = s[:, 1:2]
    acc = jnp.zeros((rpw, 32), jnp.float32)
    mx = jnp.full((rpw, 32), -jnp.inf, jnp.float32)
    for j in range(K):
        g = gath_ref[0, j * rpw:(j + 1) * rpw, :]
        hj = g[:, 0:32]
        e0 = (s0 - g[:, 32:33]) ** 2
        e1 = (s1 - g[:, 33:34]) ** 2
        w = jnp.exp(-10.0 * (e0 + e1))
        msg = hj * w
        acc = acc + msg
        mx = jnp.maximum(mx, msg)
    agg = jnp.concatenate([acc * (1.0 / K), mx], axis=1)
    h2 = _leaky((p1_ref[...] + _bdot(agg, wo2[...])) + bo2[...])
    t = h2
    for wt, bt in ((n2w1, n2b1), (n2w2, n2b2), (n2w3, n2b3)):
        t = _leaky(_bdot(t, wt[...]) + bt[...])
    ids = _bdot(t, n2w4[...]) + n2b4[...]
    u = jnp.concatenate([h2, ids], axis=1)
    for wt, bt in ((n3w1, n3b1), (n3w2, n3b2), (n3w3, n3b3)):
        u = _leaky(_bdot(u, wt[...]) + bt[...])
    p4 = x47_ref[...] + (_bdot(u, n3w4[...]) + n3b4[...])
    ids_ref[...] = ids
    p4_ref[...] = p4


def _run_d(gath, s, p1, x47, wts, nh, rpw):
    epw = K * rpw
    full = [pl.BlockSpec(w.shape, lambda i: (0,) * w.ndim) for w in wts]
    return _PCALL(
        _d_body,
        grid=(nh // rpw,),
        in_specs=[pl.BlockSpec((1, epw, DT), lambda i: (i, 0, 0)),
                  pl.BlockSpec((rpw, 2), lambda i: (i, 0)),
                  pl.BlockSpec((rpw, 32), lambda i: (i, 0)),
                  pl.BlockSpec((rpw, 4), lambda i: (i, 0))] + full,
        out_specs=[pl.BlockSpec((rpw, 6), lambda i: (i, 0)),
                   pl.BlockSpec((rpw, 4), lambda i: (i, 0))],
        out_shape=[jax.ShapeDtypeStruct((nh, 6), jnp.float32),
                   jax.ShapeDtypeStruct((nh, 4), jnp.float32)],
    )(gath, s, p1, x47, *wts)


def _wt(W):
    return jnp.transpose(W)


def _bt(b):
    return jnp.reshape(b, (1, -1))


def kernel(x, nn1, conv, nn2, nn3):
    Ws, bs, Wh, bh, Wo1, Wo2, bo2 = conv
    xp = jnp.pad(x, ((0, NPAD - N), (0, 0)))

    a_wts = []
    for (W, b) in nn1:
        a_wts += [_wt(W), _bt(b)]
    a_wts += [_wt(Ws), _bt(bs), _wt(Wh), _bt(bh), _wt(Wo1)]
    s, tab, p1 = _run_a(xp, a_wts)

    sct = jnp.transpose(s)
    x47 = xp[:, 3:7]
    nhalf = NPAD // 2
    rpwh = nhalf // NW

    gaths = []
    for h in range(2):
        lo = h * nhalf
        nbr_h = _run_b(lax.slice_in_dim(s, lo, lo + nhalf), sct, nhalf)
        nbrw_h = jnp.reshape(
            jnp.transpose(jnp.reshape(jnp.transpose(nbr_h), (K, NW, rpwh)),
                          (1, 0, 2)), (-1,))
        gaths.append(_sc_gather_fn(K * rpwh)(tab, nbrw_h))

    d_wts = [_wt(Wo2), _bt(bo2)]
    for (W, b) in nn2:
        d_wts += [_wt(W), _bt(b)]
    for (W, b) in nn3:
        d_wts += [_wt(W), _bt(b)]
    outs = []
    for h in range(2):
        lo = h * nhalf
        outs.append(_run_d(
            gaths[h],
            lax.slice_in_dim(s, lo, lo + nhalf),
            lax.slice_in_dim(p1, lo, lo + nhalf),
            lax.slice_in_dim(x47, lo, lo + nhalf),
            d_wts, nhalf, rpwh))
    ids = jnp.concatenate([outs[0][0], outs[1][0]], axis=0)
    p4 = jnp.concatenate([outs[0][1], outs[1][1]], axis=0)
    return (ids[:N], p4[:N])
